# baseline (speedup 1.0000x reference)
import functools

import jax
import jax.numpy as jnp
from jax import lax
from jax.experimental import pallas as pl
from jax.experimental.pallas import tpu as pltpu
from jax.experimental.pallas import tpu_sc as plsc

_NUM_CORES = 2
_NUM_SUBCORES = 16
_NW = _NUM_CORES * _NUM_SUBCORES
_CHUNK = 32
_NBUF = 2


@functools.lru_cache(maxsize=None)
def _make_gather(n, d):
    per_w = n // _NW
    nchunk = per_w // _CHUNK
    assert nchunk % _NBUF == 0 and nchunk >= 2 * _NBUF
    mesh = plsc.VectorSubcoreMesh(core_axis_name="c", subcore_axis_name="s")

    @functools.partial(
        pl.kernel,
        out_type=jax.ShapeDtypeStruct((n, d), jnp.float32),
        mesh=mesh,
        scratch_types=[
            pltpu.VMEM((per_w,), jnp.int32),
            pltpu.VMEM((_NBUF, _CHUNK, d), jnp.float32),
            pltpu.SemaphoreType.DMA((_NBUF,)),
            pltpu.SemaphoreType.DMA((_NBUF,)),
        ],
    )
    def k(pos_hbm, table_hbm, out_hbm, idx_v, rows_v, gsem, osem):
        wid = lax.axis_index("s") * _NUM_CORES + lax.axis_index("c")
        base = wid * per_w
        pltpu.sync_copy(pos_hbm.at[pl.ds(base, per_w)], idx_v)

        def gather_start(c, b):
            pltpu.async_copy(
                table_hbm.at[idx_v.at[pl.ds(c * _CHUNK, _CHUNK)]],
                rows_v.at[b],
                gsem.at[b],
            )

        def gather_wait(b):
            pltpu.make_async_copy(
                table_hbm.at[idx_v.at[pl.ds(0, _CHUNK)]], rows_v.at[b], gsem.at[b]
            ).wait()

        def store_start(c, b):
            pltpu.async_copy(
                rows_v.at[b], out_hbm.at[pl.ds(base + c * _CHUNK, _CHUNK)], osem.at[b]
            )

        def store_wait(b):
            pltpu.make_async_copy(
                rows_v.at[b], out_hbm.at[pl.ds(base, _CHUNK)], osem.at[b]
            ).wait()

        for b in range(_NBUF):
            gather_start(b, b)

        @pl.loop(0, nchunk - _NBUF, step=_NBUF)
        def _outer(c0):
            for b in range(_NBUF):
                gather_wait(b)
                store_start(c0 + b, b)
            for b in range(_NBUF):
                store_wait(b)
                gather_start(c0 + b + _NBUF, b)

        c0 = nchunk - _NBUF
        for b in range(_NBUF):
            gather_wait(b)
            store_start(c0 + b, b)
        for b in range(_NBUF):
            store_wait(b)

    return k


def kernel(positions, table):
    b, s = positions.shape
    n = b * s
    d = table.shape[1]
    flat = positions.reshape(n).astype(jnp.int32)
    out = _make_gather(n, d)(flat, table)
    return out.reshape(b, s, d)


# speedup vs baseline: 1.0550x; 1.0550x over previous
import functools

import jax
import jax.numpy as jnp
from jax import lax
from jax.experimental import pallas as pl
from jax.experimental.pallas import tpu as pltpu
from jax.experimental.pallas import tpu_sc as plsc

_NUM_CORES = 2
_NUM_SUBCORES = 16
_NW = _NUM_CORES * _NUM_SUBCORES
_CHUNK = 32
_NBUF = 2


@functools.lru_cache(maxsize=None)
def _make_gather(n, d):
    per_w = n // _NW
    nchunk = per_w // _CHUNK
    assert nchunk % _NBUF == 0 and nchunk >= 2 * _NBUF
    mesh = plsc.VectorSubcoreMesh(core_axis_name="c", subcore_axis_name="s")

    @functools.partial(
        pl.kernel,
        out_type=jax.ShapeDtypeStruct((n, d), jnp.float32),
        mesh=mesh,
        scratch_types=[
            pltpu.VMEM((per_w,), jnp.int32),
            pltpu.VMEM((_NBUF, _CHUNK, d), jnp.float32),
            pltpu.SemaphoreType.DMA((_NBUF,)),
            pltpu.SemaphoreType.DMA((_NBUF,)),
        ],
    )
    def k(pos_hbm, table_hbm, out_hbm, idx_v, rows_v, gsem, osem):
        wid = lax.axis_index("s") * _NUM_CORES + lax.axis_index("c")
        base = wid * per_w
        pltpu.sync_copy(pos_hbm.at[pl.ds(base, per_w)], idx_v)

        def gather_start(c, b):
            pltpu.async_copy(
                table_hbm.at[idx_v.at[pl.ds(c * _CHUNK, _CHUNK)]],
                rows_v.at[b],
                gsem.at[b],
            )

        def gather_wait(b):
            pltpu.make_async_copy(
                table_hbm.at[idx_v.at[pl.ds(0, _CHUNK)]], rows_v.at[b], gsem.at[b]
            ).wait()

        def store_start(c, b):
            pltpu.async_copy(
                rows_v.at[b], out_hbm.at[pl.ds(base + c * _CHUNK, _CHUNK)], osem.at[b]
            )

        def store_wait(b):
            pltpu.make_async_copy(
                rows_v.at[b], out_hbm.at[pl.ds(base, _CHUNK)], osem.at[b]
            ).wait()

        gs = _NBUF // 2
        nbatch = nchunk // gs

        def batch_gathers(kb, grp):
            for b in range(gs):
                gather_start(kb * gs + b, grp * gs + b)

        def batch_stores(kb, grp):
            for b in range(gs):
                gather_wait(grp * gs + b)
            for b in range(gs):
                store_start(kb * gs + b, grp * gs + b)
            for b in range(gs):
                store_wait(grp * gs + b)

        batch_gathers(0, 0)

        @pl.loop(0, nbatch - 2, step=2)
        def _outer(kb):
            batch_gathers(kb + 1, 1)
            batch_stores(kb, 0)
            batch_gathers(kb + 2, 0)
            batch_stores(kb + 1, 1)

        batch_gathers(nbatch - 1, 1)
        batch_stores(nbatch - 2, 0)
        batch_stores(nbatch - 1, 1)

    return k


def kernel(positions, table):
    b, s = positions.shape
    n = b * s
    d = table.shape[1]
    flat = positions.reshape(n).astype(jnp.int32)
    out = _make_gather(n, d)(flat, table)
    return out.reshape(b, s, d)


# speedup vs baseline: 1.0658x; 1.0102x over previous
import functools

import jax
import jax.numpy as jnp
from jax import lax
from jax.experimental import pallas as pl
from jax.experimental.pallas import tpu as pltpu
from jax.experimental.pallas import tpu_sc as plsc

_NUM_CORES = 2
_NUM_SUBCORES = 16
_NW = _NUM_CORES * _NUM_SUBCORES
_CHUNK = 8
_NBUF = 12


@functools.lru_cache(maxsize=None)
def _make_gather(n, d):
    per_w = n // _NW
    nchunk = per_w // _CHUNK
    mesh = plsc.VectorSubcoreMesh(core_axis_name="c", subcore_axis_name="s")

    @functools.partial(
        pl.kernel,
        out_type=jax.ShapeDtypeStruct((n, d), jnp.float32),
        mesh=mesh,
        scratch_types=[
            pltpu.VMEM((per_w,), jnp.int32),
            pltpu.VMEM((_NBUF, _CHUNK, d), jnp.float32),
            pltpu.SemaphoreType.DMA((_NBUF,)),
            pltpu.SemaphoreType.DMA((_NBUF,)),
        ],
    )
    def k(pos_hbm, table_hbm, out_hbm, idx_v, rows_v, gsem, osem):
        wid = lax.axis_index("s") * _NUM_CORES + lax.axis_index("c")
        base = wid * per_w
        pltpu.sync_copy(pos_hbm.at[pl.ds(base, per_w)], idx_v)

        def gather_start(c, b):
            pltpu.async_copy(
                table_hbm.at[idx_v.at[pl.ds(c * _CHUNK, _CHUNK)]],
                rows_v.at[b],
                gsem.at[b],
            )

        def gather_wait(b):
            pltpu.make_async_copy(
                table_hbm.at[idx_v.at[pl.ds(0, _CHUNK)]], rows_v.at[b], gsem.at[b]
            ).wait()

        def store_start(c, b):
            pltpu.async_copy(
                rows_v.at[b], out_hbm.at[pl.ds(base + c * _CHUNK, _CHUNK)], osem.at[b]
            )

        def store_wait(b):
            pltpu.make_async_copy(
                rows_v.at[b], out_hbm.at[pl.ds(base, _CHUNK)], osem.at[b]
            ).wait()

        gs = _NBUF // 3
        nbatch = nchunk // gs
        assert (nbatch - 2) % 3 == 0

        def batch_gathers(kb, grp):
            for b in range(gs):
                gather_start(kb * gs + b, grp * gs + b)

        def batch_stores(kb, grp):
            for b in range(gs):
                gather_wait(grp * gs + b)
            for b in range(gs):
                store_start(kb * gs + b, grp * gs + b)
            for b in range(gs):
                store_wait(grp * gs + b)

        batch_gathers(0, 0)
        batch_gathers(1, 1)

        @pl.loop(0, nbatch - 2, step=3)
        def _outer(kb):
            batch_gathers(kb + 2, 2)
            batch_stores(kb, 0)
            batch_gathers(kb + 3, 0)
            batch_stores(kb + 1, 1)
            batch_gathers(kb + 4, 1)
            batch_stores(kb + 2, 2)

        batch_stores(nbatch - 2, 0)
        batch_stores(nbatch - 1, 1)

    return k


def kernel(positions, table):
    b, s = positions.shape
    n = b * s
    d = table.shape[1]
    flat = positions.reshape(n).astype(jnp.int32)
    out = _make_gather(n, d)(flat, table)
    return out.reshape(b, s, d)
